# Initial kernel scaffold; baseline (speedup 1.0000x reference)
import jax
import jax.numpy as jnp
from jax.experimental import pallas as pl


def kernel(points, cu_seqlens, proposals, W1, b1, W2, b2, Wf, bf, cls_W, cls_b, reg_W, reg_b):
    raise NotImplementedError("write your pallas kernel here")


# speedup vs baseline: 21.9755x; 21.9755x over previous
import functools

import jax
import jax.numpy as jnp
from jax import lax
from jax.experimental import pallas as pl
from jax.experimental.pallas import tpu as pltpu

_HIDDEN = 64
_NPOS = 5
_FEAT = 256
_MIN_PTS = 4

_OFFSETS = jnp.array([[0.0, 0.0, 0.0],
                      [0.5, 0.0, 0.0],
                      [-0.5, 0.0, 0.0],
                      [0.0, 0.5, 0.0],
                      [0.0, -0.5, 0.0]], dtype=jnp.float32)

_C = 128


def _pool_body(cu_ref, prop_ref, pts_ref, weff_ref, b1p_ref, w2_ref, b2_ref,
               out_ref):
  p = pl.program_id(0)
  start = cu_ref[p]
  end = cu_ref[p + 1]
  cs = prop_ref[0]
  center = cs[:, 0:3]
  inv_den = 1.0 / (cs[:, 3:6] + 1e-6)
  nch = (end - start + (_C - 1)) // _C

  def body(i, acc):
    off = start + i * _C
    pts = pts_ref[pl.ds(off, _C), :]
    local = (pts - center) * inv_den
    base = (local[:, 0:1] * weff_ref[0:1, :]
            + local[:, 1:2] * weff_ref[1:2, :]
            + local[:, 2:3] * weff_ref[2:3, :])
    h1 = jnp.concatenate(
        [jnp.maximum(base + b1p_ref[k:k + 1, :], 0.0) for k in range(_NPOS)],
        axis=0)
    h2 = jnp.maximum(
        jnp.dot(h1, w2_ref[...], preferred_element_type=jnp.float32)
        + b2_ref[...], 0.0)
    idx = off + lax.broadcasted_iota(jnp.int32, (_C, 1), 0)
    m = idx < end
    mask5 = jnp.concatenate([m] * _NPOS, axis=0)
    h2m = jnp.where(mask5, h2, 0.0)
    mx = jnp.concatenate(
        [jnp.max(h2m[k * _C:(k + 1) * _C, :], axis=0, keepdims=True)
         for k in range(_NPOS)], axis=0)
    return jnp.maximum(acc, mx)

  acc = lax.fori_loop(0, nch, body,
                      jnp.zeros((_NPOS, _HIDDEN), dtype=jnp.float32))
  out_ref[...] = acc.reshape(1, _NPOS, _HIDDEN)


def _head_body(pool_ref, valid_ref, wf_ref, bf_ref, clsw_ref, clsb_ref,
               regw_ref, regb_ref, cls_ref, reg_ref):
  feat = (jnp.dot(pool_ref[...], wf_ref[...],
                  preferred_element_type=jnp.float32) + bf_ref[...])
  feat = feat * valid_ref[...]
  cls_ref[...] = (jnp.dot(feat, clsw_ref[...],
                          preferred_element_type=jnp.float32) + clsb_ref[...])
  reg_ref[...] = (jnp.dot(feat, regw_ref[...],
                          preferred_element_type=jnp.float32) + regb_ref[...])


@jax.jit
def kernel(points, cu_seqlens, proposals, W1, b1, W2, b2, Wf, bf, cls_W,
           cls_b, reg_W, reg_b):
  T = points.shape[0]
  P = proposals.shape[0]

  weff = W1[0:3, :] + W1[3:6, :]
  b1p = b1[None, :] - _OFFSETS @ W1[3:6, :]

  pts_pad = jnp.concatenate(
      [points, jnp.zeros((_C, 3), dtype=points.dtype)], axis=0)
  prop3 = proposals.reshape(P, 1, 6)
  cu = cu_seqlens.astype(jnp.int32)

  pooled = pl.pallas_call(
      _pool_body,
      grid=(P,),
      in_specs=[
          pl.BlockSpec(memory_space=pltpu.SMEM),
          pl.BlockSpec((1, 1, 6), lambda p: (p, 0, 0)),
          pl.BlockSpec((T + _C, 3), lambda p: (0, 0)),
          pl.BlockSpec((3, _HIDDEN), lambda p: (0, 0)),
          pl.BlockSpec((_NPOS, _HIDDEN), lambda p: (0, 0)),
          pl.BlockSpec((_HIDDEN, _HIDDEN), lambda p: (0, 0)),
          pl.BlockSpec((1, _HIDDEN), lambda p: (0, 0)),
      ],
      out_specs=pl.BlockSpec((1, _NPOS, _HIDDEN), lambda p: (p, 0, 0)),
      out_shape=jax.ShapeDtypeStruct((P, _NPOS, _HIDDEN), jnp.float32),
      compiler_params=pltpu.CompilerParams(
          dimension_semantics=("arbitrary",)),
  )(cu, prop3, pts_pad, weff, b1p, W2, b2[None, :])

  pooled2 = pooled.reshape(P, _NPOS * _HIDDEN)
  lengths = cu[1:] - cu[:-1]
  valid = (lengths >= _MIN_PTS).astype(jnp.float32).reshape(P, 1)

  cls_logits, reg_deltas = pl.pallas_call(
      _head_body,
      out_shape=(jax.ShapeDtypeStruct((P, 1), jnp.float32),
                 jax.ShapeDtypeStruct((P, 6), jnp.float32)),
  )(pooled2, valid, Wf, bf[None, :], cls_W, cls_b[None, :], reg_W,
    reg_b[None, :])

  return cls_logits, reg_deltas
